# baseline (speedup 1.0000x reference)
import functools

import jax
import jax.numpy as jnp
from jax import lax
from jax.experimental import pallas as pl
from jax.experimental.pallas import tpu as pltpu
from jax.experimental.pallas import tpu_sc as plsc

E = 160000
D = 20
F = 32
NW = 32
CHUNK = 5120
EP = NW * CHUNK
NB = 160
NBLK = CHUNK // NB
WIN = 256
GB = 64
LAM = 1.0507009873554805
LAL = 1.7580993408473766
BLK = 1024
GRID = EP // BLK


def _mesh():
    return plsc.VectorSubcoreMesh(
        core_axis_name="c", subcore_axis_name="s", num_cores=2, num_subcores=16)


def _wid():
    return lax.axis_index("s") * 2 + lax.axis_index("c")


def _gather_rows(table, prow, coff):
    packed = table.reshape(EP // 4, 4 * F)

    @functools.partial(
        pl.kernel,
        out_type=jax.ShapeDtypeStruct((EP, F), jnp.float32),
        mesh=_mesh(),
        scratch_types=[
            pltpu.VMEM((GB,), jnp.int32),
            pltpu.VMEM((GB,), jnp.int32),
            pltpu.VMEM((GB, 4 * F), jnp.float32),
            pltpu.VMEM((GB, F), jnp.float32),
            pltpu.SemaphoreType.DMA,
        ],
    )
    def k(table_hbm, prow_hbm, coff_hbm, out_hbm,
          pr_v, co_v, packed_v, rows_v, sem):
        base = _wid() * CHUNK

        def blk(b, carry):
            off = pl.multiple_of(base + b * GB, GB)
            pltpu.sync_copy(prow_hbm.at[pl.ds(off, GB)], pr_v)
            pltpu.sync_copy(coff_hbm.at[pl.ds(off, GB)], co_v)
            cpy = min(GB, 128)
            handles = [
                pltpu.async_copy(
                    table_hbm.at[pr_v.at[pl.ds(j * cpy, cpy)]],
                    packed_v.at[pl.ds(j * cpy, cpy)], sem)
                for j in range(GB // cpy)
            ]
            for h in handles:
                h.wait()

            def grp(g, c):
                goff = g * 16
                c16 = co_v[pl.ds(goff, 16)]
                for j in range(16):
                    r = goff + j
                    cj = c16[j]
                    rows_v[r, 0:16] = packed_v[r, pl.ds(cj, 16)]
                    rows_v[r, 16:32] = packed_v[r, pl.ds(cj + 16, 16)]
                return c

            lax.fori_loop(0, GB // 16, grp, 0)
            pltpu.sync_copy(rows_v, out_hbm.at[pl.ds(off, GB)])
            return carry

        lax.fori_loop(0, CHUNK // GB, blk, 0)

    return k(packed, prow, coff)


def _selu2(x0, x1):
    e0 = jnp.exp(x0) * LAL - LAL
    e1 = jnp.exp(x1) * LAL - LAL
    return (jnp.where(x0 > 0, LAM * x0, e0), jnp.where(x1 > 0, LAM * x1, e1))


def _segsum(a2, b2, prow, coff, meta):
    packed = b2.reshape(EP // 4, 4 * F)

    @functools.partial(
        pl.kernel,
        out_type=jax.ShapeDtypeStruct((EP, F), jnp.float32),
        mesh=_mesh(),
        scratch_types=[
            pltpu.VMEM((NB, F), jnp.float32),
            pltpu.VMEM((NB, F), jnp.float32),
            pltpu.VMEM((128, 4 * F), jnp.float32),
            pltpu.VMEM((128, 4 * F), jnp.float32),
            pltpu.VMEM((WIN, F), jnp.float32),
            pltpu.VMEM((WIN,), jnp.int32),
            pltpu.VMEM((WIN,), jnp.int32),
            pltpu.VMEM((3 * NB,), jnp.int32),
            pltpu.SemaphoreType.DMA,
            pltpu.SemaphoreType.DMA,
        ],
    )
    def k(a2_hbm, b2_hbm, prow_hbm, coff_hbm, meta_hbm, out_hbm,
          a_v, acc_v, pk0_v, pk1_v, bwin_v, pr_v, co_v, m_v, sem0, sem1):
        ebase = _wid() * CHUNK
        z16 = jnp.zeros((16,), jnp.float32)

        def blk(b, carry):
            off = pl.multiple_of(ebase + b * NB, NB)
            moff = pl.multiple_of(off * 3, 32)
            pltpu.sync_copy(meta_hbm.at[pl.ds(moff, 3 * NB)], m_v)
            pltpu.sync_copy(a2_hbm.at[pl.ds(off, NB)], a_v)

            def zrow(r, c):
                acc_v[r, 0:16] = z16
                acc_v[r, 16:32] = z16
                return c

            lax.fori_loop(0, NB, zrow, 0)

            lo = m_v[pl.ds(0, 16)][0]
            hi = m_v[pl.ds(2 * NB - 16, 16)][15]
            wlo0 = lo & jnp.int32(~127)
            nwin = (hi - wlo0 + (WIN - 1)) // WIN

            def wbody(wi, wcarry):
                wlo = pl.multiple_of(wlo0 + wi * WIN, 128)
                wend = wlo + WIN
                pltpu.sync_copy(prow_hbm.at[pl.ds(wlo, WIN)], pr_v)
                pltpu.sync_copy(coff_hbm.at[pl.ds(wlo, WIN)], co_v)
                bufs = [pk0_v, pk1_v]
                sems = [sem0, sem1]

                def issue(j):
                    return pltpu.async_copy(
                        b2_hbm.at[pr_v.at[pl.ds(j * 128, 128)]],
                        bufs[j % 2], sems[j % 2])

                handles = [issue(0)]
                for j in range(WIN // 128):
                    if j + 1 < WIN // 128:
                        handles.append(issue(j + 1))
                    handles[j].wait()
                    pbuf = bufs[j % 2]

                    def xgrp(g, c, j=j, pbuf=pbuf):
                        goff = g * 16
                        c16 = co_v[pl.ds(j * 128 + goff, 16)]
                        for jj in range(16):
                            r = goff + jj
                            cj = c16[jj]
                            bwin_v[j * 128 + r, 0:16] = pbuf[r, pl.ds(cj, 16)]
                            bwin_v[j * 128 + r, 16:32] = \
                                pbuf[r, pl.ds(cj + 16, 16)]
                        return c

                    lax.fori_loop(0, 8, xgrp, 0)

                def grp(g, c):
                    goff = g * 16
                    s16 = m_v[pl.ds(goff, 16)]
                    e16 = m_v[pl.ds(NB + goff, 16)]
                    p16 = m_v[pl.ds(2 * NB + goff, 16)]
                    overlap = (s16[0] < wend) & (e16[15] > wlo)

                    @pl.when(overlap)
                    def _():
                        for j in range(16):
                            r = goff + j
                            s = s16[j]
                            e = e16[j]
                            sp = p16[j]
                            p_lo = jnp.maximum(s, wlo)
                            p_hi = jnp.minimum(e, wend)

                            @pl.when(p_lo < p_hi)
                            def _():
                                a0 = a_v[r, 0:16]
                                a1 = a_v[r, 16:32]
                                acc0 = acc_v[r, 0:16]
                                acc1 = acc_v[r, 16:32]

                                def pair(p, acc):
                                    c0, c1 = acc
                                    row = p - wlo
                                    t0, t1 = _selu2(a0 + bwin_v[row, 0:16],
                                                    a1 + bwin_v[row, 16:32])
                                    return (c0 + t0, c1 + t1)

                                acc0, acc1 = lax.fori_loop(
                                    p_lo, p_hi, pair, (acc0, acc1))

                                inr = (sp >= p_lo) & (sp < p_hi)
                                m = inr.astype(jnp.float32)
                                rowx = jnp.clip(sp - wlo, 0, WIN - 1)
                                t0, t1 = _selu2(a0 + bwin_v[rowx, 0:16],
                                                a1 + bwin_v[rowx, 16:32])
                                acc_v[r, 0:16] = acc0 - m * t0
                                acc_v[r, 16:32] = acc1 - m * t1

                    return c

                lax.fori_loop(0, NB // 16, grp, 0)
                return wcarry

            lax.fori_loop(0, nwin, wbody, 0)
            pltpu.sync_copy(acc_v, out_hbm.at[pl.ds(off, NB)])
            return carry

        lax.fori_loop(0, NBLK, blk, 0)

    return k(a2, packed, prow, coff, meta)


def _full(x):
    return pl.BlockSpec(x.shape, lambda i: (0,) * x.ndim)


def _rows():
    return pl.BlockSpec((BLK, F), lambda i: (i, 0))


def _proj_body(h_ref, wa_ref, b1_ref, wb_ref, a_ref, b_ref):
    h = h_ref[...]
    a_ref[...] = jnp.dot(h, wa_ref[...],
                         preferred_element_type=jnp.float32) + b1_ref[...]
    b_ref[...] = jnp.dot(h, wb_ref[...], preferred_element_type=jnp.float32)


def _proj(h2, waT, b1m, wbT):
    return pl.pallas_call(
        _proj_body,
        grid=(GRID,),
        in_specs=[_rows(), _full(waT), _full(b1m), _full(wbT)],
        out_specs=(_rows(), _rows()),
        out_shape=(jax.ShapeDtypeStruct((EP, F), jnp.float32),
                   jax.ShapeDtypeStruct((EP, F), jnp.float32)),
    )(h2, waT, b1m, wbT)


def _gru(s2, h, nv8, w):
    agg = (jnp.dot(s2, w['w2T'], preferred_element_type=jnp.float32)
           + jnp.dot(nv8, w['b2m'], preferred_element_type=jnp.float32))
    r = jax.nn.sigmoid(jnp.dot(agg, w['wrx'], preferred_element_type=jnp.float32)
                       + jnp.dot(h, w['wrh'], preferred_element_type=jnp.float32)
                       + w['br'])
    z = jax.nn.sigmoid(jnp.dot(agg, w['wzx'], preferred_element_type=jnp.float32)
                       + jnp.dot(h, w['wzh'], preferred_element_type=jnp.float32)
                       + w['bz'])
    n = jnp.tanh(jnp.dot(agg, w['wnx'], preferred_element_type=jnp.float32)
                 + w['bin']
                 + r * (jnp.dot(h, w['wnh'], preferred_element_type=jnp.float32)
                        + w['bhn']))
    hn = (1.0 - z) * n + z * h
    mask = lax.broadcasted_iota(jnp.int32, hn.shape, 1) < D
    return jnp.where(mask, hn, 0.0)


_ROUND_KEYS = ('w2T', 'b2m', 'wrx', 'wrh', 'br', 'wzx', 'wzh', 'bz',
               'wnx', 'bin', 'wnh', 'bhn')
_FINAL_KEYS = _ROUND_KEYS + ('r1T', 'rb1m', 'r2T', 'rb2m', 'r3T', 'rb3m')


def _round_body(s2_ref, h_ref, nv_ref, *refs):
    wrefs = refs[:len(_ROUND_KEYS)]
    waT_ref, b1_ref, wbT_ref, h_out, a_out, b_out = refs[len(_ROUND_KEYS):]
    w = {kname: ref[...] for kname, ref in zip(_ROUND_KEYS, wrefs)}
    hn = _gru(s2_ref[...], h_ref[...], nv_ref[...], w)
    h_out[...] = hn
    a_out[...] = jnp.dot(hn, waT_ref[...],
                         preferred_element_type=jnp.float32) + b1_ref[...]
    b_out[...] = jnp.dot(hn, wbT_ref[...], preferred_element_type=jnp.float32)


def _round(s2, h2, nv8, w, waT, b1m, wbT):
    ws = [w[kname] for kname in _ROUND_KEYS] + [waT, b1m, wbT]
    return pl.pallas_call(
        _round_body,
        grid=(GRID,),
        in_specs=[_rows(), _rows(), pl.BlockSpec((BLK, 8), lambda i: (i, 0))]
        + [_full(x) for x in ws],
        out_specs=(_rows(), _rows(), _rows()),
        out_shape=(jax.ShapeDtypeStruct((EP, F), jnp.float32),
                   jax.ShapeDtypeStruct((EP, F), jnp.float32),
                   jax.ShapeDtypeStruct((EP, F), jnp.float32)),
    )(s2, h2, nv8, *ws)


def _selu_tc(x):
    return jnp.where(x > 0, LAM * x, LAL * jnp.exp(jnp.minimum(x, 0.0)) - LAL)


def _softplus_tc(x):
    return jnp.maximum(x, 0.0) + jnp.log1p(jnp.exp(-jnp.abs(x)))


def _final_body(s2_ref, h_ref, nv_ref, *refs):
    wrefs = refs[:len(_FINAL_KEYS)]
    out_ref = refs[len(_FINAL_KEYS)]
    w = {kname: ref[...] for kname, ref in zip(_FINAL_KEYS, wrefs)}
    hn = _gru(s2_ref[...], h_ref[...], nv_ref[...], w)
    x = _selu_tc(jnp.dot(hn, w['r1T'],
                         preferred_element_type=jnp.float32) + w['rb1m'])
    x = _selu_tc(jnp.dot(x, w['r2T'],
                         preferred_element_type=jnp.float32) + w['rb2m'])
    v = jnp.dot(x, w['r3T'], preferred_element_type=jnp.float32) + w['rb3m']
    v = _softplus_tc(v) + 0.1
    out_ref[...] = jnp.clip(v, 0.1, 10.0)


def _final(s2, h2, nv8, w):
    ws = [w[kname] for kname in _FINAL_KEYS]
    return pl.pallas_call(
        _final_body,
        grid=(GRID,),
        in_specs=[_rows(), _rows(), pl.BlockSpec((BLK, 8), lambda i: (i, 0))]
        + [_full(x) for x in ws],
        out_specs=_rows(),
        out_shape=jax.ShapeDtypeStruct((EP, F), jnp.float32),
    )(s2, h2, nv8, *ws)


def _pad_rows(x, n):
    return jnp.pad(x, ((0, n - x.shape[0]),) + ((0, 0),) * (x.ndim - 1))


def kernel(edge_index, edge_attr, W1, b1, W2, b2, Wih, Whh, bih, bhh,
           R1, rb1, R2, rb2, R3, rb3):
    src = edge_index[0]
    dst = edge_index[1]
    iE = jnp.arange(E, dtype=jnp.int32)

    order = jnp.argsort(src, stable=True).astype(jnp.int32)
    sorted_src = src[order]
    start = jnp.searchsorted(sorted_src, dst, side='left').astype(jnp.int32)
    end = jnp.searchsorted(sorted_src, dst, side='right').astype(jnp.int32)
    order2 = jnp.argsort(dst, stable=True).astype(jnp.int32)
    rank2 = jnp.zeros((E,), jnp.int32).at[order2].set(iE)
    rank_src = jnp.zeros((E,), jnp.int32).at[order].set(iE)
    start2 = start[order2]
    end2 = end[order2]
    selfloop2 = (src == dst)[order2]
    selfpos2 = jnp.where(selfloop2, rank_src[order2], jnp.int32(-1))
    nvalid2 = (end2 - start2 - selfloop2.astype(jnp.int32)).astype(jnp.float32)
    perm12 = rank2[order]

    s2a = jnp.pad(start2, (0, EP - E), constant_values=E)
    e2a = jnp.pad(end2, (0, EP - E), constant_values=E)
    spa = jnp.pad(selfpos2, (0, EP - E), constant_values=-1)
    meta = jnp.stack([s2a.reshape(-1, NB), e2a.reshape(-1, NB),
                      spa.reshape(-1, NB)], axis=1).reshape(3 * EP)
    def _split_idx(idx):
        idxp = jnp.pad(idx, (0, EP - E))
        return (idxp >> 2).astype(jnp.int32), ((idxp & 3) * F).astype(jnp.int32)

    prow_seg, coff_seg = _split_idx(perm12)
    prow0, coff0 = _split_idx(order2)
    prowF, coffF = _split_idx(rank2)
    nv8 = jnp.zeros((EP, 8), jnp.float32).at[:E, 0].set(nvalid2)

    def padT(m, rows, cols):
        mT = m.T
        return jnp.pad(mT, ((0, rows - mT.shape[0]), (0, cols - mT.shape[1])))

    waT = padT(W1[:, :D], F, F)
    wbT = padT(W1[:, D:], F, F)
    b1m = jnp.pad(b1[None, :], ((0, 0), (0, F - 32)))
    H = D
    w = {
        'w2T': padT(W2, F, F),
        'b2m': jnp.zeros((8, F), jnp.float32).at[0, :D].set(b2),
        'wrx': padT(Wih[:H], F, F),
        'wzx': padT(Wih[H:2 * H], F, F),
        'wnx': padT(Wih[2 * H:], F, F),
        'wrh': padT(Whh[:H], F, F),
        'wzh': padT(Whh[H:2 * H], F, F),
        'wnh': padT(Whh[2 * H:], F, F),
        'br': jnp.pad((bih[:H] + bhh[:H])[None, :], ((0, 0), (0, F - H))),
        'bz': jnp.pad((bih[H:2 * H] + bhh[H:2 * H])[None, :],
                      ((0, 0), (0, F - H))),
        'bin': jnp.pad(bih[2 * H:][None, :], ((0, 0), (0, F - H))),
        'bhn': jnp.pad(bhh[2 * H:][None, :], ((0, 0), (0, F - H))),
        'r1T': padT(R1, F, 64),
        'rb1m': rb1[None, :],
        'r2T': padT(R2, 64, F),
        'rb2m': rb2[None, :],
        'r3T': jnp.broadcast_to(R3.T, (32, F)),
        'rb3m': jnp.broadcast_to(rb3[None, :], (1, F)),
    }

    ea_pad = _pad_rows(jnp.pad(edge_attr, ((0, 0), (0, F - D))), EP)
    h2 = _gather_rows(ea_pad, prow0, coff0)
    a2, b2x = _proj(h2, waT, b1m, wbT)
    for t in range(4):
        s2 = _segsum(a2, b2x, prow_seg, coff_seg, meta)
        if t < 3:
            h2, a2, b2x = _round(s2, h2, nv8, w, waT, b1m, wbT)
        else:
            wmat = _final(s2, h2, nv8, w)
    out = _gather_rows(wmat, prowF, coffF)
    return out[:E, 0]


# speedup vs baseline: 1.0159x; 1.0159x over previous
import functools

import jax
import jax.numpy as jnp
from jax import lax
from jax.experimental import pallas as pl
from jax.experimental.pallas import tpu as pltpu
from jax.experimental.pallas import tpu_sc as plsc

E = 160000
D = 20
F = 32
NW = 32
CHUNK = 5120
EP = NW * CHUNK
NB = 160
NBLK = CHUNK // NB
WIN = 256
GB = 64
LAM = 1.0507009873554805
LAL = 1.7580993408473766
BLK = 1024
GRID = EP // BLK


def _mesh():
    return plsc.VectorSubcoreMesh(
        core_axis_name="c", subcore_axis_name="s", num_cores=2, num_subcores=16)


def _wid():
    return lax.axis_index("s") * 2 + lax.axis_index("c")


def _gather_rows(table, prow, coff):
    packed = table.reshape(EP // 4, 4 * F)

    @functools.partial(
        pl.kernel,
        out_type=jax.ShapeDtypeStruct((EP, F), jnp.float32),
        mesh=_mesh(),
        scratch_types=[
            pltpu.VMEM((GB,), jnp.int32),
            pltpu.VMEM((GB,), jnp.int32),
            pltpu.VMEM((GB, 4 * F), jnp.float32),
            pltpu.VMEM((GB, F), jnp.float32),
            pltpu.SemaphoreType.DMA,
        ],
    )
    def k(table_hbm, prow_hbm, coff_hbm, out_hbm,
          pr_v, co_v, packed_v, rows_v, sem):
        base = _wid() * CHUNK

        def blk(b, carry):
            off = pl.multiple_of(base + b * GB, GB)
            pltpu.sync_copy(prow_hbm.at[pl.ds(off, GB)], pr_v)
            pltpu.sync_copy(coff_hbm.at[pl.ds(off, GB)], co_v)
            cpy = min(GB, 128)
            handles = [
                pltpu.async_copy(
                    table_hbm.at[pr_v.at[pl.ds(j * cpy, cpy)]],
                    packed_v.at[pl.ds(j * cpy, cpy)], sem)
                for j in range(GB // cpy)
            ]
            for h in handles:
                h.wait()

            def grp(g, c):
                goff = g * 16
                c16 = co_v[pl.ds(goff, 16)]
                for j in range(16):
                    r = goff + j
                    cj = c16[j]
                    rows_v[r, 0:16] = packed_v[r, pl.ds(cj, 16)]
                    rows_v[r, 16:32] = packed_v[r, pl.ds(cj + 16, 16)]
                return c

            lax.fori_loop(0, GB // 16, grp, 0)
            pltpu.sync_copy(rows_v, out_hbm.at[pl.ds(off, GB)])
            return carry

        lax.fori_loop(0, CHUNK // GB, blk, 0)

    return k(packed, prow, coff)


def _selu2(x0, x1):
    e0 = jnp.exp(x0) * LAL - LAL
    e1 = jnp.exp(x1) * LAL - LAL
    return (jnp.where(x0 > 0, LAM * x0, e0), jnp.where(x1 > 0, LAM * x1, e1))


def _segsum(a2, b2, prow, coff, meta):
    packed = b2.reshape(EP // 4, 4 * F)

    @functools.partial(
        pl.kernel,
        out_type=jax.ShapeDtypeStruct((EP, F), jnp.float32),
        mesh=_mesh(),
        scratch_types=[
            pltpu.VMEM((NB, F), jnp.float32),
            pltpu.VMEM((NB, F), jnp.float32),
            pltpu.VMEM((WIN, 4 * F), jnp.float32),
            pltpu.VMEM((WIN, F), jnp.float32),
            pltpu.VMEM((WIN,), jnp.int32),
            pltpu.VMEM((WIN,), jnp.int32),
            pltpu.VMEM((3 * NB,), jnp.int32),
            pltpu.SemaphoreType.DMA,
        ],
    )
    def k(a2_hbm, b2_hbm, prow_hbm, coff_hbm, meta_hbm, out_hbm,
          a_v, acc_v, packed_v, bwin_v, pr_v, co_v, m_v, sem):
        ebase = _wid() * CHUNK
        z16 = jnp.zeros((16,), jnp.float32)

        def blk(b, carry):
            off = pl.multiple_of(ebase + b * NB, NB)
            moff = pl.multiple_of(off * 3, 32)
            pltpu.sync_copy(meta_hbm.at[pl.ds(moff, 3 * NB)], m_v)
            pltpu.sync_copy(a2_hbm.at[pl.ds(off, NB)], a_v)

            def zrow(r, c):
                acc_v[r, 0:16] = z16
                acc_v[r, 16:32] = z16
                return c

            lax.fori_loop(0, NB, zrow, 0)

            lo = m_v[pl.ds(0, 16)][0]
            hi = m_v[pl.ds(2 * NB - 16, 16)][15]
            wlo0 = lo & jnp.int32(~127)
            nwin = (hi - wlo0 + (WIN - 1)) // WIN

            def wbody(wi, wcarry):
                wlo = pl.multiple_of(wlo0 + wi * WIN, 128)
                wend = wlo + WIN
                pltpu.sync_copy(prow_hbm.at[pl.ds(wlo, WIN)], pr_v)
                pltpu.sync_copy(coff_hbm.at[pl.ds(wlo, WIN)], co_v)
                handles = [
                    pltpu.async_copy(
                        b2_hbm.at[pr_v.at[pl.ds(j * 128, 128)]],
                        packed_v.at[pl.ds(j * 128, 128)], sem)
                    for j in range(WIN // 128)
                ]
                for h in handles:
                    h.wait()

                def xgrp(g, c):
                    goff = g * 16
                    c16 = co_v[pl.ds(goff, 16)]
                    for j in range(16):
                        r = goff + j
                        cj = c16[j]
                        bwin_v[r, 0:16] = packed_v[r, pl.ds(cj, 16)]
                        bwin_v[r, 16:32] = packed_v[r, pl.ds(cj + 16, 16)]
                    return c

                lax.fori_loop(0, WIN // 16, xgrp, 0)

                def grp(g, c):
                    goff = g * 16
                    s16 = m_v[pl.ds(goff, 16)]
                    e16 = m_v[pl.ds(NB + goff, 16)]
                    p16 = m_v[pl.ds(2 * NB + goff, 16)]
                    overlap = (s16[0] < wend) & (e16[15] > wlo)

                    @pl.when(overlap)
                    def _():
                        for j in range(16):
                            r = goff + j
                            s = s16[j]
                            e = e16[j]
                            sp = p16[j]
                            p_lo = jnp.maximum(s, wlo)
                            p_hi = jnp.minimum(e, wend)

                            @pl.when(p_lo < p_hi)
                            def _():
                                a0 = a_v[r, 0:16]
                                a1 = a_v[r, 16:32]
                                acc0 = acc_v[r, 0:16]
                                acc1 = acc_v[r, 16:32]

                                def pair(p, acc):
                                    c0, c1 = acc
                                    row = p - wlo
                                    t0, t1 = _selu2(a0 + bwin_v[row, 0:16],
                                                    a1 + bwin_v[row, 16:32])
                                    return (c0 + t0, c1 + t1)

                                acc0, acc1 = lax.fori_loop(
                                    p_lo, p_hi, pair, (acc0, acc1))

                                inr = (sp >= p_lo) & (sp < p_hi)
                                m = inr.astype(jnp.float32)
                                rowx = jnp.clip(sp - wlo, 0, WIN - 1)
                                t0, t1 = _selu2(a0 + bwin_v[rowx, 0:16],
                                                a1 + bwin_v[rowx, 16:32])
                                acc_v[r, 0:16] = acc0 - m * t0
                                acc_v[r, 16:32] = acc1 - m * t1

                    return c

                lax.fori_loop(0, NB // 16, grp, 0)
                return wcarry

            lax.fori_loop(0, nwin, wbody, 0)
            pltpu.sync_copy(acc_v, out_hbm.at[pl.ds(off, NB)])
            return carry

        lax.fori_loop(0, NBLK, blk, 0)

    return k(a2, packed, prow, coff, meta)


def _full(x):
    return pl.BlockSpec(x.shape, lambda i: (0,) * x.ndim)


def _rows():
    return pl.BlockSpec((BLK, F), lambda i: (i, 0))


def _proj_body(h_ref, wa_ref, b1_ref, wb_ref, a_ref, b_ref):
    h = h_ref[...]
    a_ref[...] = jnp.dot(h, wa_ref[...],
                         preferred_element_type=jnp.float32) + b1_ref[...]
    b_ref[...] = jnp.dot(h, wb_ref[...], preferred_element_type=jnp.float32)


def _proj(h2, waT, b1m, wbT):
    return pl.pallas_call(
        _proj_body,
        grid=(GRID,),
        in_specs=[_rows(), _full(waT), _full(b1m), _full(wbT)],
        out_specs=(_rows(), _rows()),
        out_shape=(jax.ShapeDtypeStruct((EP, F), jnp.float32),
                   jax.ShapeDtypeStruct((EP, F), jnp.float32)),
    )(h2, waT, b1m, wbT)


def _gru(s2, h, nv8, w):
    agg = (jnp.dot(s2, w['w2T'], preferred_element_type=jnp.float32)
           + jnp.dot(nv8, w['b2m'], preferred_element_type=jnp.float32))
    r = jax.nn.sigmoid(jnp.dot(agg, w['wrx'], preferred_element_type=jnp.float32)
                       + jnp.dot(h, w['wrh'], preferred_element_type=jnp.float32)
                       + w['br'])
    z = jax.nn.sigmoid(jnp.dot(agg, w['wzx'], preferred_element_type=jnp.float32)
                       + jnp.dot(h, w['wzh'], preferred_element_type=jnp.float32)
                       + w['bz'])
    n = jnp.tanh(jnp.dot(agg, w['wnx'], preferred_element_type=jnp.float32)
                 + w['bin']
                 + r * (jnp.dot(h, w['wnh'], preferred_element_type=jnp.float32)
                        + w['bhn']))
    hn = (1.0 - z) * n + z * h
    mask = lax.broadcasted_iota(jnp.int32, hn.shape, 1) < D
    return jnp.where(mask, hn, 0.0)


_ROUND_KEYS = ('w2T', 'b2m', 'wrx', 'wrh', 'br', 'wzx', 'wzh', 'bz',
               'wnx', 'bin', 'wnh', 'bhn')
_FINAL_KEYS = _ROUND_KEYS + ('r1T', 'rb1m', 'r2T', 'rb2m', 'r3T', 'rb3m')


def _round_body(s2_ref, h_ref, nv_ref, *refs):
    wrefs = refs[:len(_ROUND_KEYS)]
    waT_ref, b1_ref, wbT_ref, h_out, a_out, b_out = refs[len(_ROUND_KEYS):]
    w = {kname: ref[...] for kname, ref in zip(_ROUND_KEYS, wrefs)}
    hn = _gru(s2_ref[...], h_ref[...], nv_ref[...], w)
    h_out[...] = hn
    a_out[...] = jnp.dot(hn, waT_ref[...],
                         preferred_element_type=jnp.float32) + b1_ref[...]
    b_out[...] = jnp.dot(hn, wbT_ref[...], preferred_element_type=jnp.float32)


def _round(s2, h2, nv8, w, waT, b1m, wbT):
    ws = [w[kname] for kname in _ROUND_KEYS] + [waT, b1m, wbT]
    return pl.pallas_call(
        _round_body,
        grid=(GRID,),
        in_specs=[_rows(), _rows(), pl.BlockSpec((BLK, 8), lambda i: (i, 0))]
        + [_full(x) for x in ws],
        out_specs=(_rows(), _rows(), _rows()),
        out_shape=(jax.ShapeDtypeStruct((EP, F), jnp.float32),
                   jax.ShapeDtypeStruct((EP, F), jnp.float32),
                   jax.ShapeDtypeStruct((EP, F), jnp.float32)),
    )(s2, h2, nv8, *ws)


def _selu_tc(x):
    return jnp.where(x > 0, LAM * x, LAL * jnp.exp(jnp.minimum(x, 0.0)) - LAL)


def _softplus_tc(x):
    return jnp.maximum(x, 0.0) + jnp.log1p(jnp.exp(-jnp.abs(x)))


def _final_body(s2_ref, h_ref, nv_ref, *refs):
    wrefs = refs[:len(_FINAL_KEYS)]
    out_ref = refs[len(_FINAL_KEYS)]
    w = {kname: ref[...] for kname, ref in zip(_FINAL_KEYS, wrefs)}
    hn = _gru(s2_ref[...], h_ref[...], nv_ref[...], w)
    x = _selu_tc(jnp.dot(hn, w['r1T'],
                         preferred_element_type=jnp.float32) + w['rb1m'])
    x = _selu_tc(jnp.dot(x, w['r2T'],
                         preferred_element_type=jnp.float32) + w['rb2m'])
    v = jnp.dot(x, w['r3T'], preferred_element_type=jnp.float32) + w['rb3m']
    v = _softplus_tc(v) + 0.1
    out_ref[...] = jnp.clip(v, 0.1, 10.0)


def _final(s2, h2, nv8, w):
    ws = [w[kname] for kname in _FINAL_KEYS]
    return pl.pallas_call(
        _final_body,
        grid=(GRID,),
        in_specs=[_rows(), _rows(), pl.BlockSpec((BLK, 8), lambda i: (i, 0))]
        + [_full(x) for x in ws],
        out_specs=_rows(),
        out_shape=jax.ShapeDtypeStruct((EP, F), jnp.float32),
    )(s2, h2, nv8, *ws)


def _pad_rows(x, n):
    return jnp.pad(x, ((0, n - x.shape[0]),) + ((0, 0),) * (x.ndim - 1))


def kernel(edge_index, edge_attr, W1, b1, W2, b2, Wih, Whh, bih, bhh,
           R1, rb1, R2, rb2, R3, rb3):
    src = edge_index[0]
    dst = edge_index[1]
    iE = jnp.arange(E, dtype=jnp.int32)

    order = jnp.argsort(src, stable=True).astype(jnp.int32)
    sorted_src = src[order]
    start = jnp.searchsorted(sorted_src, dst, side='left').astype(jnp.int32)
    end = jnp.searchsorted(sorted_src, dst, side='right').astype(jnp.int32)
    order2 = jnp.argsort(dst, stable=True).astype(jnp.int32)
    rank2 = jnp.zeros((E,), jnp.int32).at[order2].set(iE)
    rank_src = jnp.zeros((E,), jnp.int32).at[order].set(iE)
    start2 = start[order2]
    end2 = end[order2]
    selfloop2 = (src == dst)[order2]
    selfpos2 = jnp.where(selfloop2, rank_src[order2], jnp.int32(-1))
    nvalid2 = (end2 - start2 - selfloop2.astype(jnp.int32)).astype(jnp.float32)
    perm12 = rank2[order]

    s2a = jnp.pad(start2, (0, EP - E), constant_values=E)
    e2a = jnp.pad(end2, (0, EP - E), constant_values=E)
    spa = jnp.pad(selfpos2, (0, EP - E), constant_values=-1)
    meta = jnp.stack([s2a.reshape(-1, NB), e2a.reshape(-1, NB),
                      spa.reshape(-1, NB)], axis=1).reshape(3 * EP)
    def _split_idx(idx):
        idxp = jnp.pad(idx, (0, EP - E))
        return (idxp >> 2).astype(jnp.int32), ((idxp & 3) * F).astype(jnp.int32)

    prow_seg, coff_seg = _split_idx(perm12)
    prow0, coff0 = _split_idx(order2)
    prowF, coffF = _split_idx(rank2)
    nv8 = jnp.zeros((EP, 8), jnp.float32).at[:E, 0].set(nvalid2)

    def padT(m, rows, cols):
        mT = m.T
        return jnp.pad(mT, ((0, rows - mT.shape[0]), (0, cols - mT.shape[1])))

    waT = padT(W1[:, :D], F, F)
    wbT = padT(W1[:, D:], F, F)
    b1m = jnp.pad(b1[None, :], ((0, 0), (0, F - 32)))
    H = D
    w = {
        'w2T': padT(W2, F, F),
        'b2m': jnp.zeros((8, F), jnp.float32).at[0, :D].set(b2),
        'wrx': padT(Wih[:H], F, F),
        'wzx': padT(Wih[H:2 * H], F, F),
        'wnx': padT(Wih[2 * H:], F, F),
        'wrh': padT(Whh[:H], F, F),
        'wzh': padT(Whh[H:2 * H], F, F),
        'wnh': padT(Whh[2 * H:], F, F),
        'br': jnp.pad((bih[:H] + bhh[:H])[None, :], ((0, 0), (0, F - H))),
        'bz': jnp.pad((bih[H:2 * H] + bhh[H:2 * H])[None, :],
                      ((0, 0), (0, F - H))),
        'bin': jnp.pad(bih[2 * H:][None, :], ((0, 0), (0, F - H))),
        'bhn': jnp.pad(bhh[2 * H:][None, :], ((0, 0), (0, F - H))),
        'r1T': padT(R1, F, 64),
        'rb1m': rb1[None, :],
        'r2T': padT(R2, 64, F),
        'rb2m': rb2[None, :],
        'r3T': jnp.broadcast_to(R3.T, (32, F)),
        'rb3m': jnp.broadcast_to(rb3[None, :], (1, F)),
    }

    ea_pad = _pad_rows(jnp.pad(edge_attr, ((0, 0), (0, F - D))), EP)
    h2 = _gather_rows(ea_pad, prow0, coff0)
    a2, b2x = _proj(h2, waT, b1m, wbT)
    for t in range(4):
        s2 = _segsum(a2, b2x, prow_seg, coff_seg, meta)
        if t < 3:
            h2, a2, b2x = _round(s2, h2, nv8, w, waT, b1m, wbT)
        else:
            wmat = _final(s2, h2, nv8, w)
    out = _gather_rows(wmat, prowF, coffF)
    return out[:E, 0]
